# baseline (speedup 1.0000x reference)
import jax
import jax.numpy as jnp
from jax.experimental import pallas as pl
from jax.experimental.pallas import tpu as pltpu
from jax.experimental.pallas import tpu_sc as plsc

_NUM_CODES = 8192
_DIM = 256
_ROWS = 9216
_R = 256
_GW = 128
_SUBS = 32
_IDS_PER_SUB = _ROWS // _SUBS


def _vq_tc_kernel(z_ref, w_ref, znorm_ref, cnorm_ref, ids_ref):
    z2 = z_ref[...] * 2.0
    w = w_ref[...]
    mm2 = jax.lax.dot_general(
        z2, w, (((1,), (1,)), ((), ())),
        preferred_element_type=jnp.float32)
    d = (znorm_ref[...] + cnorm_ref[...]) - mm2

    lane = jax.lax.broadcasted_iota(jnp.int32, (_R, 128), 1)
    val = d[:, 0:128]
    colacc = lane
    for g in range(1, _NUM_CODES // 128):
        dg = d[:, 128 * g:128 * (g + 1)]
        better = dg < val
        val = jnp.where(better, dg, val)
        colacc = jnp.where(better, lane + 128 * g, colacc)
    rowmin = jnp.min(val, axis=1, keepdims=True)
    cand = jnp.where(val == rowmin, colacc, jnp.int32(_NUM_CODES))
    ids_ref[...] = jnp.min(cand, axis=1)[None, None, :]


def _sc_gather(W, ids_row):
    mesh = plsc.VectorSubcoreMesh(core_axis_name="core",
                                  subcore_axis_name="subcore")

    @pl.kernel(out_type=jax.ShapeDtypeStruct((_ROWS, _DIM), jnp.float32),
               mesh=mesh)
    def gather_kernel(w_hbm, i_hbm, o_hbm):
        def body(i_vmem, o_vmem):
            pltpu.sync_copy(w_hbm.at[i_vmem.at[0]], o_vmem)

        pltpu.emit_pipeline(
            body,
            grid=(_ROWS // _GW,),
            in_specs=[pl.BlockSpec((1, _GW), index_map=lambda i: (0, i))],
            out_specs=[pl.BlockSpec((_GW, _DIM), index_map=lambda i: (i, 0))],
            core_axis_name=("core", "subcore"),
            dimension_semantics=(pltpu.PARALLEL,),
        )(i_hbm, o_hbm)

    return gather_kernel(W, ids_row)


def _sc_gather_hist(W, ids_row, nrows):
    mesh = plsc.VectorSubcoreMesh(core_axis_name="core",
                                  subcore_axis_name="subcore")

    nwin = nrows // _GW

    @pl.kernel(
        out_type=[jax.ShapeDtypeStruct((nrows, _DIM), jnp.float32),
                  jax.ShapeDtypeStruct((_SUBS, _NUM_CODES), jnp.int32)],
        mesh=mesh)
    def gather_kernel(w_hbm, i_hbm, one0_hbm, o_hbm, part_hbm):
        def body(i_vmem, o_vmem):
            pltpu.sync_copy(w_hbm.at[i_vmem.at[0]], o_vmem)

        pltpu.emit_pipeline(
            body,
            grid=(nwin,),
            in_specs=[pl.BlockSpec((1, _GW), index_map=lambda i: (0, i))],
            out_specs=[pl.BlockSpec((_GW, _DIM), index_map=lambda i: (i, 0))],
            core_axis_name=("core", "subcore"),
            dimension_semantics=(pltpu.PARALLEL,),
        )(i_hbm, o_hbm)

        u = (jax.lax.axis_index("core") * 16
             + jax.lax.axis_index("subcore"))

        def hist_scope(bins_vmem, ids_vmem, one0_vmem, sem):
            pltpu.async_copy(one0_hbm.at[0], one0_vmem, sem).wait()
            one0 = one0_vmem[...]

            @pl.loop(0, (_NUM_CODES + 16) // 16)
            def _(k):
                bins_vmem[pl.ds(k * 16, 16)] = jnp.zeros((16,), jnp.int32)

            for w_i in range((nwin + _SUBS - 1) // _SUBS):
                w = u + _SUBS * w_i

                @pl.when(w < nwin)
                def _():
                    off = pl.multiple_of(w * _GW, _GW)
                    pltpu.async_copy(i_hbm.at[0, pl.ds(off, _GW)],
                                     ids_vmem, sem).wait()

                    @pl.loop(0, _GW // 16)
                    def _(kk):
                        v = ids_vmem[pl.ds(kk * 16, 16)]
                        for k in range(16):
                            idx = v[k]
                            cur = bins_vmem[pl.ds(idx, 16)]
                            bins_vmem[pl.ds(idx, 16)] = cur + one0

            pltpu.async_copy(bins_vmem.at[pl.ds(0, _NUM_CODES)],
                             part_hbm.at[u], sem).wait()

        pl.run_scoped(hist_scope,
                      pltpu.VMEM((_NUM_CODES + 16,), jnp.int32),
                      pltpu.VMEM((_GW,), jnp.int32),
                      pltpu.VMEM((16,), jnp.int32),
                      pltpu.SemaphoreType.DMA)

    one0_arr = jnp.zeros((1, 16), jnp.int32).at[0, 0].set(1)
    return gather_kernel(W, ids_row, one0_arr)


def _tc_ids(zf_c, W, znorm_c, cnorm):
    tiles = zf_c.shape[0] // _R
    return pl.pallas_call(
        _vq_tc_kernel,
        grid=(tiles,),
        in_specs=[
            pl.BlockSpec((_R, _DIM), lambda i: (i, 0)),
            pl.BlockSpec((_NUM_CODES, _DIM), lambda i: (0, 0)),
            pl.BlockSpec((_R, 1), lambda i: (i, 0)),
            pl.BlockSpec((1, _NUM_CODES), lambda i: (0, 0)),
        ],
        out_specs=pl.BlockSpec((1, 1, _R), lambda i: (i, 0, 0)),
        out_shape=jax.ShapeDtypeStruct((tiles, 1, _R), jnp.int32),
    )(zf_c, W, znorm_c, cnorm)


_HALF = _ROWS // 2


def kernel(z, W):
    B, S, D = z.shape
    zf = z.reshape(-1, D)
    znorm = jnp.sum(zf ** 2, axis=1, keepdims=True)
    cnorm = jnp.sum(W ** 2, axis=1)[None, :]

    ids_a = _tc_ids(zf[:_HALF], W, znorm[:_HALF], cnorm)
    g_a, p_a = _sc_gather_hist(W, ids_a.reshape(1, _HALF), _HALF)
    ids_b = _tc_ids(zf[_HALF:], W, znorm[_HALF:], cnorm)
    g_b, p_b = _sc_gather_hist(W, ids_b.reshape(1, _HALF), _HALF)

    gathered = jnp.concatenate([g_a, g_b], axis=0)
    zq = z + (gathered.reshape(z.shape) - z)

    counts = (jnp.sum(p_a, axis=0)
              + jnp.sum(p_b, axis=0)).astype(jnp.float32)
    avg_probs = counts / (B * S)
    perplexity = jnp.exp(-jnp.sum(avg_probs * jnp.log(avg_probs + 1e-10)))
    return (zq, jnp.asarray(0.0, dtype=jnp.float32), perplexity)


# speedup vs baseline: 1.1398x; 1.1398x over previous
import jax
import jax.numpy as jnp
from jax.experimental import pallas as pl
from jax.experimental.pallas import tpu as pltpu
from jax.experimental.pallas import tpu_sc as plsc

_NUM_CODES = 8192
_DIM = 256
_ROWS = 9216
_R = 256
_GW = 128
_SUBS = 32
_IDS_PER_SUB = _ROWS // _SUBS


def _vq_tc_kernel(z_ref, w_ref, znorm_ref, cnorm_ref, ids_ref):
    z2 = z_ref[...] * 2.0
    w = w_ref[...]
    mm2 = jax.lax.dot_general(
        z2, w, (((1,), (1,)), ((), ())),
        preferred_element_type=jnp.float32)
    d = (znorm_ref[...] + cnorm_ref[...]) - mm2

    lane = jax.lax.broadcasted_iota(jnp.int32, (_R, 128), 1)
    val = d[:, 0:128]
    colacc = lane
    for g in range(1, _NUM_CODES // 128):
        dg = d[:, 128 * g:128 * (g + 1)]
        better = dg < val
        val = jnp.where(better, dg, val)
        colacc = jnp.where(better, lane + 128 * g, colacc)
    rowmin = jnp.min(val, axis=1, keepdims=True)
    cand = jnp.where(val == rowmin, colacc, jnp.int32(_NUM_CODES))
    ids_ref[...] = jnp.min(cand, axis=1)[None, None, :]


def _sc_gather(W, ids_row):
    mesh = plsc.VectorSubcoreMesh(core_axis_name="core",
                                  subcore_axis_name="subcore")

    @pl.kernel(out_type=jax.ShapeDtypeStruct((_ROWS, _DIM), jnp.float32),
               mesh=mesh)
    def gather_kernel(w_hbm, i_hbm, o_hbm):
        def body(i_vmem, o_vmem):
            pltpu.sync_copy(w_hbm.at[i_vmem.at[0]], o_vmem)

        pltpu.emit_pipeline(
            body,
            grid=(_ROWS // _GW,),
            in_specs=[pl.BlockSpec((1, _GW), index_map=lambda i: (0, i))],
            out_specs=[pl.BlockSpec((_GW, _DIM), index_map=lambda i: (i, 0))],
            core_axis_name=("core", "subcore"),
            dimension_semantics=(pltpu.PARALLEL,),
        )(i_hbm, o_hbm)

    return gather_kernel(W, ids_row)


_NWIN = _ROWS // _GW


def _sc_gather_hist(W, ids_row):
    mesh = plsc.VectorSubcoreMesh(core_axis_name="core",
                                  subcore_axis_name="subcore")

    @pl.kernel(
        out_type=[jax.ShapeDtypeStruct((_ROWS, _DIM), jnp.float32),
                  jax.ShapeDtypeStruct((_SUBS, _NUM_CODES), jnp.int32)],
        mesh=mesh)
    def gather_kernel(w_hbm, i_hbm, one0_hbm, o_hbm, part_hbm):
        def body(i_vmem, o_vmem):
            pltpu.sync_copy(w_hbm.at[i_vmem.at[0]], o_vmem)

        pltpu.emit_pipeline(
            body,
            grid=(_NWIN,),
            in_specs=[pl.BlockSpec((1, _GW), index_map=lambda i: (0, i))],
            out_specs=[pl.BlockSpec((_GW, _DIM), index_map=lambda i: (i, 0))],
            core_axis_name=("core", "subcore"),
            dimension_semantics=(pltpu.PARALLEL,),
        )(i_hbm, o_hbm)

        u = (jax.lax.axis_index("core") * 16
             + jax.lax.axis_index("subcore"))

        def hist_scope(bins_vmem, ids_vmem, one0_vmem, sem):
            pltpu.async_copy(one0_hbm.at[0], one0_vmem, sem).wait()
            one0 = one0_vmem[...]

            @pl.loop(0, (_NUM_CODES + 16) // 16)
            def _(k):
                bins_vmem[pl.ds(k * 16, 16)] = jnp.zeros((16,), jnp.int32)

            for w_i in range((_NWIN + _SUBS - 1) // _SUBS):
                w = u + _SUBS * w_i

                @pl.when(w < _NWIN)
                def _():
                    off = pl.multiple_of(w * _GW, _GW)
                    pltpu.async_copy(i_hbm.at[0, pl.ds(off, _GW)],
                                     ids_vmem, sem).wait()

                    @pl.loop(0, _GW // 16)
                    def _(kk):
                        v = ids_vmem[pl.ds(kk * 16, 16)]
                        for k in range(16):
                            idx = v[k]
                            cur = bins_vmem[pl.ds(idx, 16)]
                            bins_vmem[pl.ds(idx, 16)] = cur + one0

            pltpu.async_copy(bins_vmem.at[pl.ds(0, _NUM_CODES)],
                             part_hbm.at[u], sem).wait()

        pl.run_scoped(hist_scope,
                      pltpu.VMEM((_NUM_CODES + 16,), jnp.int32),
                      pltpu.VMEM((_GW,), jnp.int32),
                      pltpu.VMEM((16,), jnp.int32),
                      pltpu.SemaphoreType.DMA)

    one0_arr = jnp.zeros((1, 16), jnp.int32).at[0, 0].set(1)
    return gather_kernel(W, ids_row, one0_arr)


def kernel(z, W):
    B, S, D = z.shape
    zf = z.reshape(-1, D)
    znorm = jnp.sum(zf ** 2, axis=1, keepdims=True)
    cnorm = jnp.sum(W ** 2, axis=1)[None, :]

    ids3 = pl.pallas_call(
        _vq_tc_kernel,
        grid=(_ROWS // _R,),
        in_specs=[
            pl.BlockSpec((_R, _DIM), lambda i: (i, 0)),
            pl.BlockSpec((_NUM_CODES, _DIM), lambda i: (0, 0)),
            pl.BlockSpec((_R, 1), lambda i: (i, 0)),
            pl.BlockSpec((1, _NUM_CODES), lambda i: (0, 0)),
        ],
        out_specs=pl.BlockSpec((1, 1, _R), lambda i: (i, 0, 0)),
        out_shape=jax.ShapeDtypeStruct((_ROWS // _R, 1, _R), jnp.int32),
    )(zf, W, znorm, cnorm)

    gathered, partials = _sc_gather_hist(W, ids3.reshape(1, _ROWS))
    zq = z + (gathered.reshape(z.shape) - z)

    counts = jnp.sum(partials, axis=0).astype(jnp.float32)
    avg_probs = counts / (B * S)
    perplexity = jnp.exp(-jnp.sum(avg_probs * jnp.log(avg_probs + 1e-10)))
    return (zq, jnp.asarray(0.0, dtype=jnp.float32), perplexity)


# speedup vs baseline: 1.1870x; 1.0414x over previous
import jax
import jax.numpy as jnp
from jax.experimental import pallas as pl
from jax.experimental.pallas import tpu as pltpu
from jax.experimental.pallas import tpu_sc as plsc

_NUM_CODES = 8192
_DIM = 256
_ROWS = 9216
_R = 512
_GW = 128
_SUBS = 32
_IDS_PER_SUB = _ROWS // _SUBS


def _vq_tc_kernel(z_ref, w_ref, znorm_ref, cnorm_ref, ids_ref):
    z2 = z_ref[...] * 2.0
    w = w_ref[...]
    mm2 = jax.lax.dot_general(
        z2, w, (((1,), (1,)), ((), ())),
        preferred_element_type=jnp.float32)
    d = (znorm_ref[...] + cnorm_ref[...]) - mm2

    lane = jax.lax.broadcasted_iota(jnp.int32, (_R, 128), 1)
    val = d[:, 0:128]
    colacc = lane
    for g in range(1, _NUM_CODES // 128):
        dg = d[:, 128 * g:128 * (g + 1)]
        better = dg < val
        val = jnp.where(better, dg, val)
        colacc = jnp.where(better, lane + 128 * g, colacc)
    rowmin = jnp.min(val, axis=1, keepdims=True)
    cand = jnp.where(val == rowmin, colacc, jnp.int32(_NUM_CODES))
    ids_ref[...] = jnp.min(cand, axis=1)[None, None, :]


def _sc_gather(W, ids_row):
    mesh = plsc.VectorSubcoreMesh(core_axis_name="core",
                                  subcore_axis_name="subcore")

    @pl.kernel(out_type=jax.ShapeDtypeStruct((_ROWS, _DIM), jnp.float32),
               mesh=mesh)
    def gather_kernel(w_hbm, i_hbm, o_hbm):
        def body(i_vmem, o_vmem):
            pltpu.sync_copy(w_hbm.at[i_vmem.at[0]], o_vmem)

        pltpu.emit_pipeline(
            body,
            grid=(_ROWS // _GW,),
            in_specs=[pl.BlockSpec((1, _GW), index_map=lambda i: (0, i))],
            out_specs=[pl.BlockSpec((_GW, _DIM), index_map=lambda i: (i, 0))],
            core_axis_name=("core", "subcore"),
            dimension_semantics=(pltpu.PARALLEL,),
        )(i_hbm, o_hbm)

    return gather_kernel(W, ids_row)


_NWIN = _ROWS // _GW


def _sc_gather_hist(W, ids_row):
    mesh = plsc.VectorSubcoreMesh(core_axis_name="core",
                                  subcore_axis_name="subcore")

    @pl.kernel(
        out_type=[jax.ShapeDtypeStruct((_ROWS, _DIM), jnp.float32),
                  jax.ShapeDtypeStruct((_SUBS, _NUM_CODES), jnp.int32)],
        mesh=mesh)
    def gather_kernel(w_hbm, i_hbm, one0_hbm, o_hbm, part_hbm):
        def body(i_vmem, o_vmem):
            pltpu.sync_copy(w_hbm.at[i_vmem.at[0]], o_vmem)

        pltpu.emit_pipeline(
            body,
            grid=(_NWIN,),
            in_specs=[pl.BlockSpec((1, _GW), index_map=lambda i: (0, i))],
            out_specs=[pl.BlockSpec((_GW, _DIM), index_map=lambda i: (i, 0))],
            core_axis_name=("core", "subcore"),
            dimension_semantics=(pltpu.PARALLEL,),
        )(i_hbm, o_hbm)

        u = (jax.lax.axis_index("core") * 16
             + jax.lax.axis_index("subcore"))

        def hist_scope(bins_vmem, ids_vmem, one0_vmem, sem):
            pltpu.async_copy(one0_hbm.at[0], one0_vmem, sem).wait()
            one0 = one0_vmem[...]

            @pl.loop(0, (_NUM_CODES + 16) // 16)
            def _(k):
                bins_vmem[pl.ds(k * 16, 16)] = jnp.zeros((16,), jnp.int32)

            for w_i in range((_NWIN + _SUBS - 1) // _SUBS):
                w = u + _SUBS * w_i

                @pl.when(w < _NWIN)
                def _():
                    off = pl.multiple_of(w * _GW, _GW)
                    pltpu.async_copy(i_hbm.at[0, pl.ds(off, _GW)],
                                     ids_vmem, sem).wait()

                    @pl.loop(0, _GW // 16)
                    def _(kk):
                        v = ids_vmem[pl.ds(kk * 16, 16)]
                        for k in range(16):
                            idx = v[k]
                            cur = bins_vmem[pl.ds(idx, 16)]
                            bins_vmem[pl.ds(idx, 16)] = cur + one0

            pltpu.async_copy(bins_vmem.at[pl.ds(0, _NUM_CODES)],
                             part_hbm.at[u], sem).wait()

        pl.run_scoped(hist_scope,
                      pltpu.VMEM((_NUM_CODES + 16,), jnp.int32),
                      pltpu.VMEM((_GW,), jnp.int32),
                      pltpu.VMEM((16,), jnp.int32),
                      pltpu.SemaphoreType.DMA)

    one0_arr = jnp.zeros((1, 16), jnp.int32).at[0, 0].set(1)
    return gather_kernel(W, ids_row, one0_arr)


def kernel(z, W):
    B, S, D = z.shape
    zf = z.reshape(-1, D)
    znorm = jnp.sum(zf ** 2, axis=1, keepdims=True)
    cnorm = jnp.sum(W ** 2, axis=1)[None, :]

    ids3 = pl.pallas_call(
        _vq_tc_kernel,
        grid=(_ROWS // _R,),
        in_specs=[
            pl.BlockSpec((_R, _DIM), lambda i: (i, 0)),
            pl.BlockSpec((_NUM_CODES, _DIM), lambda i: (0, 0)),
            pl.BlockSpec((_R, 1), lambda i: (i, 0)),
            pl.BlockSpec((1, _NUM_CODES), lambda i: (0, 0)),
        ],
        out_specs=pl.BlockSpec((1, 1, _R), lambda i: (i, 0, 0)),
        out_shape=jax.ShapeDtypeStruct((_ROWS // _R, 1, _R), jnp.int32),
    )(zf, W, znorm, cnorm)

    gathered, partials = _sc_gather_hist(W, ids3.reshape(1, _ROWS))
    zq = z + (gathered.reshape(z.shape) - z)

    counts = jnp.sum(partials, axis=0).astype(jnp.float32)
    avg_probs = counts / (B * S)
    perplexity = jnp.exp(-jnp.sum(avg_probs * jnp.log(avg_probs + 1e-10)))
    return (zq, jnp.asarray(0.0, dtype=jnp.float32), perplexity)
